# baseline (speedup 1.0000x reference)
import jax
import jax.numpy as jnp
from jax.experimental import pallas as pl
from jax.experimental.pallas import tpu as pltpu

_N_BINS = 256
_TAU = 0.01
_EPS = 1e-10
_LANE = 128
_SUB = 32

_LOG2E = 1.4426950408889634
_C = -_LOG2E / (2.0 * _TAU * _TAU * 255.0 * 255.0)


def _weights(t_row, iota_bins):
    d = t_row - iota_bins
    return jnp.exp2((d * d) * _C)


def _hist_kernel(x_ref, hist_ref):
    i = pl.program_id(1)

    @pl.when(i == 0)
    def _():
        hist_ref[...] = jnp.zeros_like(hist_ref)

    t = x_ref[0] * 255.0
    iota_bins = jax.lax.broadcasted_iota(
        jnp.int32, (_N_BINS, _LANE), 0).astype(jnp.float32)
    acc = hist_ref[0]
    for k in range(_SUB):
        acc = acc + _weights(t[k : k + 1, :], iota_bins)
    hist_ref[0] = acc


def _eq_kernel(x_ref, hist_ref, out_ref, cdf_ref):
    i = pl.program_id(1)

    @pl.when(i == 0)
    def _():
        h = hist_ref[0]
        ones_sq = jnp.ones((_LANE, _LANE), jnp.float32)
        h_rep = jnp.dot(h, ones_sq, preferred_element_type=jnp.float32)
        r = jax.lax.broadcasted_iota(jnp.int32, (_N_BINS, _N_BINS), 0)
        c = jax.lax.broadcasted_iota(jnp.int32, (_N_BINS, _N_BINS), 1)
        tri_low = (r >= c).astype(jnp.float32)
        cdf = jnp.dot(tri_low, h_rep, preferred_element_type=jnp.float32)
        total = cdf[_N_BINS - 1 :, :]
        cdf = cdf * (1.0 / (total + _EPS))
        c0 = cdf[0:1, :]
        cdf_ref[...] = (cdf - c0) * (1.0 / (1.0 - c0 + _EPS))

    t = x_ref[0] * 255.0
    iota_bins = jax.lax.broadcasted_iota(
        jnp.int32, (_N_BINS, _LANE), 0).astype(jnp.float32)
    cdf_rep = cdf_ref[...]
    for k in range(_SUB):
        w = _weights(t[k : k + 1, :], iota_bins)
        den = jnp.sum(w, axis=0, keepdims=True)
        num = jnp.sum(w * cdf_rep, axis=0, keepdims=True)
        out_ref[0, k : k + 1, :] = num / (den + _EPS)


def kernel(x):
    B, _, H, W = x.shape
    hw_rows = (H * W) // _LANE
    nc = hw_rows // _SUB
    x3 = x.reshape(B, hw_rows, _LANE)

    hist = pl.pallas_call(
        _hist_kernel,
        grid=(B, nc),
        in_specs=[pl.BlockSpec((1, _SUB, _LANE), lambda b, i: (b, i, 0))],
        out_specs=pl.BlockSpec((1, _N_BINS, _LANE), lambda b, i: (b, 0, 0)),
        out_shape=jax.ShapeDtypeStruct((B, _N_BINS, _LANE), jnp.float32),
        compiler_params=pltpu.CompilerParams(
            dimension_semantics=("parallel", "arbitrary")),
    )(x3)

    out = pl.pallas_call(
        _eq_kernel,
        grid=(B, nc),
        in_specs=[
            pl.BlockSpec((1, _SUB, _LANE), lambda b, i: (b, i, 0)),
            pl.BlockSpec((1, _N_BINS, _LANE), lambda b, i: (b, 0, 0)),
        ],
        out_specs=pl.BlockSpec((1, _SUB, _LANE), lambda b, i: (b, i, 0)),
        out_shape=jax.ShapeDtypeStruct((B, hw_rows, _LANE), jnp.float32),
        scratch_shapes=[pltpu.VMEM((_N_BINS, _LANE), jnp.float32)],
        compiler_params=pltpu.CompilerParams(
            dimension_semantics=("parallel", "arbitrary")),
    )(x3, hist)

    return out.reshape(B, 1, H, W)


# speedup vs baseline: 1.5865x; 1.5865x over previous
import jax
import jax.numpy as jnp
from jax.experimental import pallas as pl
from jax.experimental.pallas import tpu as pltpu

_N_BINS = 256
_TAU = 0.01
_EPS = 1e-10
_LANE = 128
_SUB = 32

_LOG2E = 1.4426950408889634
_C = -_LOG2E / (2.0 * _TAU * _TAU * 255.0 * 255.0)


def _weights(t_row, iota_bins):
    d = t_row - iota_bins
    return jnp.exp2((d * d) * _C)


def _hist_kernel(x_ref, hist_ref):
    i = pl.program_id(1)

    @pl.when(i == 0)
    def _():
        hist_ref[...] = jnp.zeros_like(hist_ref)

    t = x_ref[0] * 255.0
    iota_bins = jax.lax.broadcasted_iota(
        jnp.int32, (_N_BINS, _LANE), 0).astype(jnp.float32)
    acc = hist_ref[0]
    for k in range(_SUB):
        acc = acc + _weights(t[k : k + 1, :], iota_bins)
    hist_ref[0] = acc


_SEGS = 16
_RES = _SEGS * _LANE
_TROWS = _SEGS + 1


def _eq_kernel(x_ref, hist_ref, out_ref, ftab_ref, dtab_ref):
    i = pl.program_id(1)

    @pl.when(i == 0)
    def _():
        h = hist_ref[0]
        ones_sq = jnp.ones((_LANE, _LANE), jnp.float32)
        h_rep = jnp.dot(h, ones_sq, preferred_element_type=jnp.float32)
        r = jax.lax.broadcasted_iota(jnp.int32, (_N_BINS, _N_BINS), 0)
        c = jax.lax.broadcasted_iota(jnp.int32, (_N_BINS, _N_BINS), 1)
        tri_low = (r >= c).astype(jnp.float32)
        cdf = jnp.dot(tri_low, h_rep, preferred_element_type=jnp.float32)
        total = cdf[_N_BINS - 1 :, :]
        cdf = cdf * (1.0 / (total + _EPS))
        c0 = cdf[0:1, :]
        cdf_n = (cdf - c0) * (1.0 / (1.0 - c0 + _EPS))

        iota_bins = jax.lax.broadcasted_iota(
            jnp.int32, (_N_BINS, _LANE), 0).astype(jnp.float32)
        lane = jax.lax.broadcasted_iota(
            jnp.int32, (1, _LANE), 1).astype(jnp.float32)
        rows = []
        for s in range(_TROWS):
            tg = (lane + float(s * _LANE)) * (255.0 / _RES)
            w = _weights(tg, iota_bins)
            den = jnp.sum(w, axis=0, keepdims=True)
            num = jnp.sum(w * cdf_n, axis=0, keepdims=True)
            rows.append(num / (den + _EPS))
        ftab = jnp.concatenate(rows, axis=0)
        down = jnp.concatenate([ftab[1:, :], ftab[:1, :]], axis=0)
        fnext = jnp.concatenate([ftab[:, 1:], down[:, 0:1]], axis=1)
        ftab_ref[...] = ftab
        dtab_ref[...] = fnext - ftab

    x = x_ref[0]
    u = x * float(_RES)
    fl = jnp.floor(u)
    idx = jnp.minimum(fl.astype(jnp.int32), _RES - 1)
    frac = u - fl
    hi = jax.lax.shift_right_logical(idx, 7)
    lo = jax.lax.bitwise_and(idx, _LANE - 1)
    fg = jnp.zeros_like(x)
    dg = jnp.zeros_like(x)
    for s in range(_SEGS):
        tf = jnp.broadcast_to(ftab_ref[s : s + 1, :], (_SUB, _LANE))
        td = jnp.broadcast_to(dtab_ref[s : s + 1, :], (_SUB, _LANE))
        gf = jnp.take_along_axis(tf, lo, axis=1)
        gd = jnp.take_along_axis(td, lo, axis=1)
        m = hi == s
        fg = jnp.where(m, gf, fg)
        dg = jnp.where(m, gd, dg)
    out_ref[0] = fg + dg * frac


def kernel(x):
    B, _, H, W = x.shape
    hw_rows = (H * W) // _LANE
    nc = hw_rows // _SUB
    x3 = x.reshape(B, hw_rows, _LANE)

    hist = pl.pallas_call(
        _hist_kernel,
        grid=(B, nc),
        in_specs=[pl.BlockSpec((1, _SUB, _LANE), lambda b, i: (b, i, 0))],
        out_specs=pl.BlockSpec((1, _N_BINS, _LANE), lambda b, i: (b, 0, 0)),
        out_shape=jax.ShapeDtypeStruct((B, _N_BINS, _LANE), jnp.float32),
        compiler_params=pltpu.CompilerParams(
            dimension_semantics=("parallel", "arbitrary")),
    )(x3)

    out = pl.pallas_call(
        _eq_kernel,
        grid=(B, nc),
        in_specs=[
            pl.BlockSpec((1, _SUB, _LANE), lambda b, i: (b, i, 0)),
            pl.BlockSpec((1, _N_BINS, _LANE), lambda b, i: (b, 0, 0)),
        ],
        out_specs=pl.BlockSpec((1, _SUB, _LANE), lambda b, i: (b, i, 0)),
        out_shape=jax.ShapeDtypeStruct((B, hw_rows, _LANE), jnp.float32),
        scratch_shapes=[pltpu.VMEM((_TROWS, _LANE), jnp.float32),
                        pltpu.VMEM((_TROWS, _LANE), jnp.float32)],
        compiler_params=pltpu.CompilerParams(
            dimension_semantics=("parallel", "arbitrary")),
    )(x3, hist)

    return out.reshape(B, 1, H, W)


# speedup vs baseline: 8.4273x; 5.3117x over previous
import functools
import math

import jax
import jax.numpy as jnp
from jax.experimental import pallas as pl
from jax.experimental.pallas import tpu as pltpu

_N_BINS = 256
_TAU = 0.01
_EPS = 1e-10
_LANE = 128
_SUBE = 2048
_CHUNK = 64

_LOG2E = 1.4426950408889634
_C = -_LOG2E / (2.0 * _TAU * _TAU * 255.0 * 255.0)

_MMAX = 32
_PERIOD = 320.0
_SIG = _TAU * 255.0
_A0 = math.sqrt(2.0 * math.pi) * _SIG / _PERIOD
_Q = 0.5 * (2.0 * math.pi * _SIG / _PERIOD) ** 2

_PC = (
    -0.8090169943749322,
    -3.6931636609766265,
    15.969355376461667,
    24.300042880200195,
    -52.53707336600454,
    -47.96636190677093,
    69.13601722930966,
    45.08656784713891,
    -48.73893305426519,
    -24.721364277085346,
    21.37919001988334,
    8.87097106997191,
    -6.393040601579775,
    -2.235144874011542,
    1.3805613970684725,
    0.3837390925861489,
    -0.20711425807719946,
)
_PS = (
    0.5877852522924749,
    -5.0832036923094055,
    -11.602415825843154,
    33.44613968699522,
    38.170418099311,
    -66.02003332034546,
    -50.23025684833626,
    62.056336879079474,
    35.41091084834041,
    -34.026039515276764,
    -15.532939347783374,
    12.209850498468024,
    4.645219130085321,
    -3.07644262527986,
    -1.0047665179119578,
    0.5282268006411842,
    0.15347294185827037,
)


def _horner(coefs, y):
    acc = jnp.full_like(y, coefs[-1])
    for c in coefs[-2::-1]:
        acc = acc * y + c
    return acc


def _fold8(v):
    parts = [v[r : r + 8] for r in range(0, _CHUNK, 8)]
    while len(parts) > 1:
        parts = [a + b for a, b in zip(parts[0::2], parts[1::2])]
    return parts[0]


def _weights(t_row, iota_bins):
    d = t_row - iota_bins
    return jnp.exp2((d * d) * _C)


_SEGS = 1
_RES = _SEGS * _LANE
_TROWS = _SEGS + 1


def _fused_kernel(hwpx, nce, x_ref, out_ref, mom_ref, ftab_ref, dtab_ref):
    i = pl.program_id(1)

    @pl.when(i == 0)
    def _():
        mom_ref[...] = jnp.zeros_like(mom_ref)

    @pl.when(i < nce)
    def _():
        phi = x_ref[0] * (255.0 / _PERIOD)
        y = phi - 0.4
        for ch in range(0, _SUBE, _CHUNK):
            cc = _horner(_PC, y[ch : ch + _CHUNK])
            ss = _horner(_PS, y[ch : ch + _CHUNK])
            twoc = cc + cc
            cp = jnp.ones_like(cc)
            sp = jnp.zeros_like(ss)
            for mm in range(_MMAX):
                mom_ref[mm] = mom_ref[mm] + _fold8(cc)
                mom_ref[_MMAX + mm] = mom_ref[_MMAX + mm] + _fold8(ss)
                if mm < _MMAX - 1:
                    cn = twoc * cc - cp
                    sn = twoc * ss - sp
                    cp, cc = cc, cn
                    sp, ss = ss, sn

    @pl.when(i == nce)
    def _():
        mom = mom_ref[...]
        m2 = mom.reshape(2 * _MMAX * 8, _LANE)
        ones_sq = jnp.ones((_LANE, _LANE), jnp.float32)
        lsum = jnp.dot(m2, ones_sq, preferred_element_type=jnp.float32)
        g = jnp.sum(lsum.reshape(2 * _MMAX, 8, _LANE), axis=1)
        cs = g[0:_MMAX]
        sn = g[_MMAX:]
        jr = jax.lax.broadcasted_iota(
            jnp.int32, (_N_BINS, _MMAX), 0).astype(jnp.float32)
        mc = jax.lax.broadcasted_iota(
            jnp.int32, (_N_BINS, _MMAX), 1).astype(jnp.float32) + 1.0
        ang = jr * mc * (2.0 * math.pi / _PERIOD)
        amr = (2.0 * _A0) * jnp.exp2((-_Q * _LOG2E) * (mc * mc))
        kc = jnp.cos(ang) * amr
        ks = jnp.sin(ang) * amr
        hist_rep = (jnp.dot(kc, cs, preferred_element_type=jnp.float32)
                    + jnp.dot(ks, sn, preferred_element_type=jnp.float32)
                    + (_A0 * hwpx))
        r = jax.lax.broadcasted_iota(jnp.int32, (_N_BINS, _N_BINS), 0)
        c = jax.lax.broadcasted_iota(jnp.int32, (_N_BINS, _N_BINS), 1)
        tri_low = (r >= c).astype(jnp.float32)
        cdf = jnp.dot(tri_low, hist_rep, preferred_element_type=jnp.float32)
        total = cdf[_N_BINS - 1 :, :]
        cdf = cdf * (1.0 / (total + _EPS))
        c0 = cdf[0:1, :]
        cdf_n = (cdf - c0) * (1.0 / (1.0 - c0 + _EPS))

        iota_bins = jax.lax.broadcasted_iota(
            jnp.int32, (_N_BINS, _LANE), 0).astype(jnp.float32)
        lane = jax.lax.broadcasted_iota(
            jnp.int32, (1, _LANE), 1).astype(jnp.float32)
        rows = []
        for s in range(_TROWS):
            tg = (lane + float(s * _LANE)) * (255.0 / _RES)
            w = _weights(tg, iota_bins)
            den = jnp.sum(w, axis=0, keepdims=True)
            num = jnp.sum(w * cdf_n, axis=0, keepdims=True)
            rows.append(num / (den + _EPS))
        ftab = jnp.concatenate(rows, axis=0)
        down = jnp.concatenate([ftab[1:, :], ftab[:1, :]], axis=0)
        fnext = jnp.concatenate([ftab[:, 1:], down[:, 0:1]], axis=1)
        ftab_ref[...] = ftab
        dtab_ref[...] = fnext - ftab

    @pl.when(i >= nce)
    def _():
        x = x_ref[0]
        u = x * float(_RES)
        fl = jnp.floor(u)
        idx = jnp.minimum(fl.astype(jnp.int32), _RES - 1)
        frac = u - fl
        hi = jax.lax.shift_right_logical(idx, 7)
        lo = jax.lax.bitwise_and(idx, _LANE - 1)
        fg = jnp.zeros_like(x)
        dg = jnp.zeros_like(x)
        for s in range(_SEGS):
            tf = jnp.broadcast_to(ftab_ref[s : s + 1, :], (_SUBE, _LANE))
            td = jnp.broadcast_to(dtab_ref[s : s + 1, :], (_SUBE, _LANE))
            gf = jnp.take_along_axis(tf, lo, axis=1)
            gd = jnp.take_along_axis(td, lo, axis=1)
            m = hi == s
            fg = jnp.where(m, gf, fg)
            dg = jnp.where(m, gd, dg)
        out_ref[0] = fg + dg * frac


def kernel(x):
    B, _, H, W = x.shape
    hw_rows = (H * W) // _LANE
    nce = hw_rows // _SUBE
    x3 = x.reshape(B, hw_rows, _LANE)

    out = pl.pallas_call(
        functools.partial(_fused_kernel, float(H * W), nce),
        grid=(B, 2 * nce),
        in_specs=[
            pl.BlockSpec(
                (1, _SUBE, _LANE),
                lambda b, i: (b, jnp.where(i < nce, i, i - nce), 0)),
        ],
        out_specs=pl.BlockSpec(
            (1, _SUBE, _LANE),
            lambda b, i: (b, jnp.maximum(i - nce, 0), 0)),
        out_shape=jax.ShapeDtypeStruct((B, hw_rows, _LANE), jnp.float32),
        scratch_shapes=[pltpu.VMEM((2 * _MMAX, 8, _LANE), jnp.float32),
                        pltpu.VMEM((_TROWS, _LANE), jnp.float32),
                        pltpu.VMEM((_TROWS, _LANE), jnp.float32)],
        compiler_params=pltpu.CompilerParams(
            dimension_semantics=("parallel", "arbitrary")),
    )(x3)

    return out.reshape(B, 1, H, W)
